# baseline (speedup 1.0000x reference)
import functools

import jax
import jax.numpy as jnp
from jax import lax
from jax.experimental import pallas as pl
from jax.experimental.pallas import tpu as pltpu


def _router_body(num_alt, k_top, x_ref, alt_ref, altemb_ref, w1_ref, b1_ref,
                 wg1a_ref, wg1b_ref, bg1_ref, wg2_ref, bg2c_ref,
                 gw_ref, idx_ref):
    x = x_ref[...]
    blk = x.shape[0]
    f = jnp.maximum(
        jnp.dot(x, w1_ref[...], preferred_element_type=jnp.float32)
        + b1_ref[...], 0.0)

    t2 = jnp.dot(altemb_ref[...], wg1b_ref[...],
                 preferred_element_type=jnp.float32)
    t2_hi = t2.astype(jnp.bfloat16).astype(jnp.float32)
    t2_lo = t2 - t2_hi
    aid = alt_ref[...]
    oh = (aid == lax.broadcasted_iota(jnp.int32, (blk, 8), 1)
          ).astype(jnp.float32)
    acc = (jnp.dot(f, wg1a_ref[...], preferred_element_type=jnp.float32)
           + jnp.dot(oh, t2_hi, preferred_element_type=jnp.float32)
           + jnp.dot(oh, t2_lo, preferred_element_type=jnp.float32)
           + bg1_ref[...])
    h = jnp.maximum(acc, 0.0)

    logits_t = lax.dot_general(
        wg2_ref[...], h, (((0,), (1,)), ((), ())),
        preferred_element_type=jnp.float32) + bg2c_ref[...]

    e = jnp.exp(logits_t)
    s = jnp.sum(e, axis=0, keepdims=True)
    gw_t = e * (1.0 / s)
    gw_ref[...] = gw_t.T

    E = gw_t.shape[0]
    iota = lax.broadcasted_iota(jnp.int32, (E, blk), 0)
    work = gw_t
    rows = []
    for _ in range(k_top):
        mxk = jnp.max(work, axis=0, keepdims=True)
        cand = jnp.where(work == mxk, iota, E)
        sel = jnp.min(cand, axis=0, keepdims=True)
        rows.append(sel)
        work = jnp.where(cand == sel, -1.0, work)
    idx_t = jnp.concatenate(rows, axis=0)
    idx_ref[...] = idx_t.T


def kernel(feat_stats, alt_idx, alt_embed, W1, b1, Wg1, bg1, Wg2, bg2):
    B, D = feat_stats.shape
    num_alt, H = alt_embed.shape
    E = Wg2.shape[1]
    K = 8
    blk = 4096

    Wg1a = Wg1[:H]
    Wg1b = Wg1[H:]
    alt_pad = jnp.zeros((8, H), jnp.float32).at[:num_alt].set(alt_embed)
    alt2d = alt_idx.astype(jnp.int32).reshape(B, 1)
    b1r = b1.reshape(1, H)
    bg1r = bg1.reshape(1, H)
    bg2c = bg2.reshape(E, 1)

    grid = (B // blk,)
    row = lambda i: (i, 0)
    rep = lambda i: (0, 0)
    gw, idx = pl.pallas_call(
        functools.partial(_router_body, num_alt, K),
        grid=grid,
        compiler_params=pltpu.CompilerParams(
            dimension_semantics=("parallel",)),
        in_specs=[
            pl.BlockSpec((blk, D), row),
            pl.BlockSpec((blk, 1), row),
            pl.BlockSpec((8, H), rep),
            pl.BlockSpec((D, H), rep),
            pl.BlockSpec((1, H), rep),
            pl.BlockSpec((H, H), rep),
            pl.BlockSpec((H, H), rep),
            pl.BlockSpec((1, H), rep),
            pl.BlockSpec((H, E), rep),
            pl.BlockSpec((E, 1), rep),
        ],
        out_specs=[
            pl.BlockSpec((blk, E), row),
            pl.BlockSpec((blk, K), row),
        ],
        out_shape=[
            jax.ShapeDtypeStruct((B, E), jnp.float32),
            jax.ShapeDtypeStruct((B, K), jnp.int32),
        ],
    )(feat_stats, alt2d, alt_pad, W1, b1r, Wg1a, Wg1b, bg1r, Wg2, bg2c)
    return gw, idx


# speedup vs baseline: 1.1593x; 1.1593x over previous
import functools

import jax
import jax.numpy as jnp
from jax import lax
from jax.experimental import pallas as pl
from jax.experimental.pallas import tpu as pltpu


def _router_body(num_alt, k_top, x_ref, alt_ref, altemb_ref, w1_ref, b1_ref,
                 wg1_ref, bg1_ref, wg2_ref, bg2c_ref,
                 gw_ref, idx_ref):
    x = x_ref[...]
    blk = x.shape[0]
    H = w1_ref.shape[1]
    f = jnp.maximum(
        jnp.dot(x, w1_ref[...], preferred_element_type=jnp.float32)
        + b1_ref[...], 0.0)

    t2 = jnp.dot(altemb_ref[...], wg1_ref[H:, :],
                 preferred_element_type=jnp.float32)
    t2_hi = t2.astype(jnp.bfloat16).astype(jnp.float32)
    t2_lo = t2 - t2_hi
    aid = alt_ref[...].reshape(1, blk)
    oh_t = (aid == lax.broadcasted_iota(jnp.int32, (8, blk), 0)
            ).astype(jnp.float32)
    acc = (jnp.dot(f, wg1_ref[:H, :], preferred_element_type=jnp.float32)
           + lax.dot_general(oh_t, t2_hi, (((0,), (0,)), ((), ())),
                             preferred_element_type=jnp.float32)
           + lax.dot_general(oh_t, t2_lo, (((0,), (0,)), ((), ())),
                             preferred_element_type=jnp.float32)
           + bg1_ref[...])
    h = jnp.maximum(acc, 0.0)

    logits_t = lax.dot_general(
        wg2_ref[...], h, (((0,), (1,)), ((), ())),
        preferred_element_type=jnp.float32) + bg2c_ref[...]

    e = jnp.exp(logits_t)
    s = jnp.sum(e, axis=0, keepdims=True)
    gw_t = e * (1.0 / s)
    gw_ref[...] = gw_t.T

    E = gw_t.shape[0]
    iota = lax.broadcasted_iota(jnp.int32, (E, blk), 0)
    work = gw_t
    rows = []
    for _ in range(k_top):
        mxk = jnp.max(work, axis=0, keepdims=True)
        cand = jnp.where(work == mxk, iota, E)
        sel = jnp.min(cand, axis=0, keepdims=True)
        rows.append(sel)
        work = jnp.where(cand == sel, -1.0, work)
    idx_t = jnp.concatenate(rows, axis=0)
    idx_ref[...] = idx_t.T


def kernel(feat_stats, alt_idx, alt_embed, W1, b1, Wg1, bg1, Wg2, bg2):
    B, D = feat_stats.shape
    num_alt, H = alt_embed.shape
    E = Wg2.shape[1]
    K = 8
    blk = 4096
    nb = B // blk

    alt_pad = jnp.zeros((8, H), jnp.float32).at[:num_alt].set(alt_embed)
    alt3d = alt_idx.astype(jnp.int32).reshape(nb, 1, blk)
    b1r = b1.reshape(1, H)
    bg1r = bg1.reshape(1, H)
    bg2c = bg2.reshape(E, 1)

    row = lambda i: (i, 0)
    rep = lambda i: (0, 0)
    gw, idx = pl.pallas_call(
        functools.partial(_router_body, num_alt, K),
        grid=(nb,),
        compiler_params=pltpu.CompilerParams(
            dimension_semantics=("parallel",)),
        in_specs=[
            pl.BlockSpec((blk, D), row),
            pl.BlockSpec((1, 1, blk), lambda i: (i, 0, 0)),
            pl.BlockSpec((8, H), rep),
            pl.BlockSpec((D, H), rep),
            pl.BlockSpec((1, H), rep),
            pl.BlockSpec((2 * H, H), rep),
            pl.BlockSpec((1, H), rep),
            pl.BlockSpec((H, E), rep),
            pl.BlockSpec((E, 1), rep),
        ],
        out_specs=[
            pl.BlockSpec((blk, E), row),
            pl.BlockSpec((blk, K), row),
        ],
        out_shape=[
            jax.ShapeDtypeStruct((B, E), jnp.float32),
            jax.ShapeDtypeStruct((B, K), jnp.int32),
        ],
    )(feat_stats, alt3d, alt_pad, W1, b1r, Wg1, bg1r, Wg2, bg2c)
    return gw, idx


# speedup vs baseline: 1.1976x; 1.0330x over previous
import functools

import jax
import jax.numpy as jnp
from jax import lax
from jax.experimental import pallas as pl
from jax.experimental.pallas import tpu as pltpu


def _router_body(num_alt, k_top, x_ref, alt_ref, altemb_ref, w1_ref, b1_ref,
                 wg1_ref, bg1_ref, wg2_ref, bg2_ref,
                 gw_ref, idx_ref):
    x = x_ref[...]
    blk = x.shape[0]
    H = w1_ref.shape[1]
    f = jnp.maximum(
        jnp.dot(x, w1_ref[...], preferred_element_type=jnp.float32)
        + b1_ref[...].reshape(1, H), 0.0)

    t2 = jnp.dot(altemb_ref[...], wg1_ref[H:, :],
                 preferred_element_type=jnp.float32)
    t2_hi = t2.astype(jnp.bfloat16).astype(jnp.float32)
    t2_lo = t2 - t2_hi
    aid = alt_ref[...].reshape(1, blk)
    oh_t = (aid == lax.broadcasted_iota(jnp.int32, (num_alt, blk), 0)
            ).astype(jnp.float32)
    acc = (jnp.dot(f, wg1_ref[:H, :], preferred_element_type=jnp.float32)
           + lax.dot_general(oh_t, t2_hi, (((0,), (0,)), ((), ())),
                             preferred_element_type=jnp.float32)
           + lax.dot_general(oh_t, t2_lo, (((0,), (0,)), ((), ())),
                             preferred_element_type=jnp.float32)
           + bg1_ref[...].reshape(1, H))
    h = jnp.maximum(acc, 0.0)

    E = wg2_ref.shape[1]
    logits_t = (lax.dot_general(wg2_ref[...], h, (((0,), (1,)), ((), ())),
                                preferred_element_type=jnp.float32)
                + bg2_ref[...].reshape(E, 1))

    e = jnp.exp(logits_t)
    s = jnp.sum(e, axis=0, keepdims=True)
    gw_t = e * (1.0 / s)
    gw_ref[...] = gw_t.T

    iota = lax.broadcasted_iota(jnp.int32, (E, blk), 0)
    work = gw_t
    rows = []
    for _ in range(k_top):
        mxk = jnp.max(work, axis=0, keepdims=True)
        cand = jnp.where(work == mxk, iota, E)
        sel = jnp.min(cand, axis=0, keepdims=True)
        rows.append(sel)
        work = jnp.where(cand == sel, -1.0, work)
    idx_t = jnp.concatenate(rows, axis=0)
    idx_ref[...] = idx_t.T


def kernel(feat_stats, alt_idx, alt_embed, W1, b1, Wg1, bg1, Wg2, bg2):
    B, D = feat_stats.shape
    num_alt, H = alt_embed.shape
    E = Wg2.shape[1]
    K = 8
    blk = 4096
    nb = B // blk

    alt32 = alt_idx.astype(jnp.int32)

    row = lambda i: (i, 0)
    rep = lambda i: (0, 0)
    gw, idx = pl.pallas_call(
        functools.partial(_router_body, num_alt, K),
        grid=(nb,),
        compiler_params=pltpu.CompilerParams(
            dimension_semantics=("parallel",)),
        in_specs=[
            pl.BlockSpec((blk, D), row),
            pl.BlockSpec((blk,), lambda i: (i,)),
            pl.BlockSpec((num_alt, H), rep),
            pl.BlockSpec((D, H), rep),
            pl.BlockSpec((H,), lambda i: (0,)),
            pl.BlockSpec((2 * H, H), rep),
            pl.BlockSpec((H,), lambda i: (0,)),
            pl.BlockSpec((H, E), rep),
            pl.BlockSpec((E,), lambda i: (0,)),
        ],
        out_specs=[
            pl.BlockSpec((blk, E), row),
            pl.BlockSpec((blk, K), row),
        ],
        out_shape=[
            jax.ShapeDtypeStruct((B, E), jnp.float32),
            jax.ShapeDtypeStruct((B, K), jnp.int32),
        ],
    )(feat_stats, alt32, alt_embed, W1, b1, Wg1, bg1, Wg2, bg2)
    return gw, idx


# speedup vs baseline: 1.6943x; 1.4148x over previous
import functools

import jax
import jax.numpy as jnp
from jax import lax
from jax.experimental import pallas as pl
from jax.experimental.pallas import tpu as pltpu


def _router_body(num_alt, k_top, x_ref, alt_ref, altemb_ref, w1_ref, b1_ref,
                 wg1_ref, bg1_ref, wg2_ref, bg2_ref,
                 gw_ref, idx_ref):
    x = x_ref[...]
    blk = x.shape[0]
    H = w1_ref.shape[1]
    f = jnp.maximum(
        jnp.dot(x, w1_ref[...], preferred_element_type=jnp.float32)
        + b1_ref[...].reshape(1, H), 0.0)

    t2 = jnp.dot(altemb_ref[...], wg1_ref[H:, :],
                 preferred_element_type=jnp.float32)
    t2_hi = t2.astype(jnp.bfloat16).astype(jnp.float32)
    t2_lo = t2 - t2_hi
    aid = alt_ref[...].reshape(1, blk)
    oh_t = (aid == lax.broadcasted_iota(jnp.int32, (num_alt, blk), 0)
            ).astype(jnp.float32)
    acc = (jnp.dot(f, wg1_ref[:H, :], preferred_element_type=jnp.float32)
           + lax.dot_general(oh_t, t2_hi, (((0,), (0,)), ((), ())),
                             preferred_element_type=jnp.float32)
           + lax.dot_general(oh_t, t2_lo, (((0,), (0,)), ((), ())),
                             preferred_element_type=jnp.float32)
           + bg1_ref[...].reshape(1, H))
    h = jnp.maximum(acc, 0.0)

    E = wg2_ref.shape[1]
    logits_t = (lax.dot_general(wg2_ref[...], h, (((0,), (1,)), ((), ())),
                                preferred_element_type=jnp.float32)
                + bg2_ref[...].reshape(E, 1))

    e = jnp.exp(logits_t)
    s = jnp.sum(e, axis=0, keepdims=True)
    gw_t = e * (1.0 / s)
    gw_ref[...] = gw_t

    iota = lax.broadcasted_iota(jnp.int32, (E, blk), 0)
    work = gw_t
    rows = []
    for _ in range(k_top):
        mxk = jnp.max(work, axis=0, keepdims=True)
        cand = jnp.where(work == mxk, iota, E)
        sel = jnp.min(cand, axis=0, keepdims=True)
        rows.append(sel)
        work = jnp.where(cand == sel, -1.0, work)
    idx_ref[...] = jnp.concatenate(rows, axis=0)


def kernel(feat_stats, alt_idx, alt_embed, W1, b1, Wg1, bg1, Wg2, bg2):
    B, D = feat_stats.shape
    num_alt, H = alt_embed.shape
    E = Wg2.shape[1]
    K = 8
    blk = 4096
    nb = B // blk

    alt32 = alt_idx.astype(jnp.int32)

    row = lambda i: (i, 0)
    rep = lambda i: (0, 0)
    gw, idx = pl.pallas_call(
        functools.partial(_router_body, num_alt, K),
        grid=(nb,),
        compiler_params=pltpu.CompilerParams(
            dimension_semantics=("parallel",)),
        in_specs=[
            pl.BlockSpec((blk, D), row),
            pl.BlockSpec((blk,), lambda i: (i,)),
            pl.BlockSpec((num_alt, H), rep),
            pl.BlockSpec((D, H), rep),
            pl.BlockSpec((H,), lambda i: (0,)),
            pl.BlockSpec((2 * H, H), rep),
            pl.BlockSpec((H,), lambda i: (0,)),
            pl.BlockSpec((H, E), rep),
            pl.BlockSpec((E,), lambda i: (0,)),
        ],
        out_specs=[
            pl.BlockSpec((E, blk), lambda i: (0, i)),
            pl.BlockSpec((K, blk), lambda i: (0, i)),
        ],
        out_shape=[
            jax.ShapeDtypeStruct((E, B), jnp.float32),
            jax.ShapeDtypeStruct((K, B), jnp.int32),
        ],
    )(feat_stats, alt32, alt_embed, W1, b1, Wg1, bg1, Wg2, bg2)
    return gw.T, idx.T
